# baseline (speedup 1.0000x reference)
import math

import jax
import jax.numpy as jnp
import numpy as np
from jax import lax
from jax.experimental import pallas as pl
from jax.experimental.pallas import tpu as pltpu
from jax.experimental.pallas import tpu_sc as plsc

GRID_RES = 128
FEATURE_DIM = 28
NUM_VOXELS = GRID_RES ** 3
NUM_RAYS = 4096
NUM_SAMPLES = 50
RAY_LENGTH = 100.0
DELTA = RAY_LENGTH / NUM_SAMPLES
THETA = 0.5
PHI = 0.3

PROJ_DIM = 8
NUM_NBR = 8
S64 = 64

VPR = 16
N_PROWS = NUM_VOXELS // VPR

N_GATHER = NUM_RAYS * NUM_NBR * S64
N_GROWS = N_GATHER // 16

NUM_WORKERS = 32
PER_W = N_GATHER // NUM_WORKERS
CHUNK = 4096
N_CHUNKS = PER_W // CHUNK


def _sh_vec():
    y00 = 0.5 * math.sqrt(1.0 / math.pi)
    h3 = 0.5 * math.sqrt(3.0 / math.pi)
    q5 = 0.25 * math.sqrt(5.0 / math.pi)
    h15 = 0.5 * math.sqrt(15.0 / math.pi)
    q15 = 0.25 * math.sqrt(15.0 / math.pi)
    st, ct = math.sin(THETA), math.cos(THETA)
    sp, cp = math.sin(PHI), math.cos(PHI)
    return np.array([
        y00,
        h3 * st * sp,
        h3 * ct,
        h3 * st * cp,
        h15 * st * cp * st * sp,
        h15 * st * sp * ct,
        q5 * (3.0 * ct * ct - 1.0),
        h15 * st * cp * ct,
        q15 * ((st * cp) ** 2 - (st * sp) ** 2),
    ], dtype=np.float32)


def _proj_matrix():
    Y = _sh_vec()
    M = np.zeros((FEATURE_DIM, PROJ_DIM), dtype=np.float32)
    M[0, 0] = 1.0
    for c in range(3):
        M[1 + c * 9:1 + (c + 1) * 9, 1 + c] = Y
    return M


def _proj_body(t_ref, m_ref, b_ref, o_ref):
    o_ref[...] = jnp.dot(t_ref[...], m_ref[...],
                         preferred_element_type=jnp.float32) + b_ref[...]


def _project(table):
    M = _proj_matrix()
    D = np.zeros((VPR * FEATURE_DIM, 128), dtype=np.float32)
    for j in range(VPR):
        D[j * FEATURE_DIM:(j + 1) * FEATURE_DIM,
          j * PROJ_DIM:j * PROJ_DIM + PROJ_DIM] = M
    bias = np.zeros((1, 128), dtype=np.float32)
    for j in range(VPR):
        bias[0, j * PROJ_DIM + 7] = 1.0
    t448 = table.reshape(N_PROWS, VPR * FEATURE_DIM)
    BLK = 2048
    grid = N_PROWS // BLK
    return pl.pallas_call(
        _proj_body,
        grid=(grid,),
        in_specs=[
            pl.BlockSpec((BLK, VPR * FEATURE_DIM), lambda i: (i, 0)),
            pl.BlockSpec((VPR * FEATURE_DIM, 128), lambda i: (0, 0)),
            pl.BlockSpec((1, 128), lambda i: (0, 0)),
        ],
        out_specs=pl.BlockSpec((BLK, 128), lambda i: (i, 0)),
        out_shape=jax.ShapeDtypeStruct((N_PROWS, 128), jnp.float32),
        compiler_params=pltpu.CompilerParams(
            allow_input_fusion=[True, False, False]),
    )(t448, jnp.asarray(D), jnp.asarray(bias))


def _gather_body(p_hbm, idx_hbm, out_hbm,
                 idx_v0, idx_v1, rows_v0, rows_v1, sem0, sem1):
    wid = lax.axis_index("s") * 2 + lax.axis_index("c")
    idx_v = (idx_v0, idx_v1)
    rows_v = (rows_v0, rows_v1)
    sems = (sem0, sem1)
    base0 = wid * PER_W
    pltpu.sync_copy(idx_hbm.at[pl.ds(base0, CHUNK)], idx_v[0])
    handles = [pltpu.async_copy(p_hbm.at[idx_v[0]], rows_v[0], sems[0])]
    for c in range(1, N_CHUNKS + 1):
        if c < N_CHUNKS:
            b = c % 2
            base = wid * PER_W + c * CHUNK
            pltpu.sync_copy(idx_hbm.at[pl.ds(base, CHUNK)], idx_v[b])
            handles.append(
                pltpu.async_copy(p_hbm.at[idx_v[b]], rows_v[b], sems[b]))
        pb = (c - 1) % 2
        pbase = wid * PER_W + (c - 1) * CHUNK
        handles[c - 1].wait()
        pltpu.sync_copy(rows_v[pb], out_hbm.at[pl.ds(pbase, CHUNK)])


def _sc_gather(P2, idx_flat):
    mesh = plsc.VectorSubcoreMesh(core_axis_name="c", subcore_axis_name="s")
    k = pl.kernel(
        _gather_body,
        mesh=mesh,
        out_type=jax.ShapeDtypeStruct((N_GATHER, PROJ_DIM), jnp.float32),
        scratch_types=[
            pltpu.VMEM((CHUNK,), jnp.int32),
            pltpu.VMEM((CHUNK,), jnp.int32),
            pltpu.VMEM((CHUNK, PROJ_DIM), jnp.float32),
            pltpu.VMEM((CHUNK, PROJ_DIM), jnp.float32),
            pltpu.SemaphoreType.DMA,
            pltpu.SemaphoreType.DMA,
        ],
        compiler_params=pltpu.CompilerParams(use_tc_tiling_on_sc=False),
    )
    return k(P2, idx_flat)


R_BLK = 128
BR = R_BLK * 32


def _fused_consts():
    E = np.zeros((16, 128), dtype=np.float32)
    for t in range(16):
        E[t, t * 8:t * 8 + 8] = 1.0
    SPR7 = np.zeros((128, 128), dtype=np.float32)
    for t in range(16):
        for f in range(8):
            SPR7[t * 8 + 7, t * 8 + f] = 1.0
    LTIN = np.zeros((128, 128), dtype=np.float32)
    for t in range(16):
        for t2 in range(t, 16):
            LTIN[t * 8, t2 * 8] = 1.0
    TOTM = np.zeros((128, 128), dtype=np.float32)
    for t in range(16):
        for t2 in range(16):
            TOTM[t * 8, t2 * 8] = 1.0
    SPRD = np.zeros((128, 128), dtype=np.float32)
    for t in range(16):
        for c in (1, 2, 3):
            SPRD[t * 8, t * 8 + c] = 1.0
    UC = np.zeros((128, 3), dtype=np.float32)
    for t in range(16):
        for c in range(3):
            UC[t * 8 + 1 + c, c] = 1.0
    SEL = np.zeros((R_BLK, BR), dtype=np.float32)
    for r in range(R_BLK):
        for q in range(4):
            SEL[r, r * 32 + q] = 1.0
    return E, SPR7, LTIN, TOTM, SPRD, UC, SEL


def _fused_body(g_ref, w_ref, e_ref, spr7_ref, ltin_ref, totm_ref,
                sprd_ref, uc_ref, sel_ref, o_ref):
    g = g_ref[...]
    w = w_ref[...]
    wexp = jnp.dot(w, e_ref[...], preferred_element_type=jnp.float32)
    acc = g * wexp
    acc = acc + pltpu.roll(acc, BR - 4, axis=0)
    acc = acc + pltpu.roll(acc, BR - 8, axis=0)
    acc = acc + pltpu.roll(acc, BR - 16, axis=0)
    nrm = jnp.dot(acc, spr7_ref[...], preferred_element_type=jnp.float32)
    z = acc / nrm
    rowi = lax.broadcasted_iota(jnp.int32, (BR, 128), 0)
    lanei = lax.broadcasted_iota(jnp.int32, (BR, 128), 1)
    dmask = (lanei % 8 == 0).astype(jnp.float32)
    taud = jnp.maximum(z, 0.0) * dmask * DELTA
    pre = jnp.dot(taud, ltin_ref[...], preferred_element_type=jnp.float32)
    excl = pre - taud
    tot = jnp.dot(taud, totm_ref[...], preferred_element_type=jnp.float32)
    base = jnp.zeros_like(tot)
    for k in (1, 2, 3):
        mk = (rowi % 32 >= k).astype(jnp.float32)
        base = base + pltpu.roll(tot, k, axis=0) * mk
    T = jnp.exp(-(excl + base))
    alpha = 1.0 - jnp.exp(-taud)
    wta = T * alpha * dmask
    wta8 = jnp.dot(wta, sprd_ref[...], preferred_element_type=jnp.float32)
    smask = ((rowi % 32) * 16 + lanei // 8 < NUM_SAMPLES).astype(jnp.float32)
    contrib = z * wta8 * smask
    ruc = jnp.dot(contrib, uc_ref[...], preferred_element_type=jnp.float32)
    o_ref[...] = jnp.dot(sel_ref[...], ruc,
                         preferred_element_type=jnp.float32)


def _fused(G128, w16):
    E, SPR7, LTIN, TOTM, SPRD, UC, SEL = _fused_consts()
    grid = NUM_RAYS // R_BLK
    return pl.pallas_call(
        _fused_body,
        grid=(grid,),
        in_specs=[
            pl.BlockSpec((BR, 128), lambda i: (i, 0)),
            pl.BlockSpec((BR, 16), lambda i: (i, 0)),
            pl.BlockSpec((16, 128), lambda i: (0, 0)),
            pl.BlockSpec((128, 128), lambda i: (0, 0)),
            pl.BlockSpec((128, 128), lambda i: (0, 0)),
            pl.BlockSpec((128, 128), lambda i: (0, 0)),
            pl.BlockSpec((128, 128), lambda i: (0, 0)),
            pl.BlockSpec((128, 3), lambda i: (0, 0)),
            pl.BlockSpec((R_BLK, BR), lambda i: (0, 0)),
        ],
        out_specs=pl.BlockSpec((R_BLK, 3), lambda i: (i, 0)),
        out_shape=jax.ShapeDtypeStruct((NUM_RAYS, 3), jnp.float32),
    )(G128, w16, jnp.asarray(E), jnp.asarray(SPR7), jnp.asarray(LTIN),
      jnp.asarray(TOTM), jnp.asarray(SPRD), jnp.asarray(UC), jnp.asarray(SEL))


def kernel(weights, table, indices):
    idx_t = jnp.transpose(indices.astype(jnp.int32), (0, 2, 1))
    idx_p = jnp.pad(idx_t, ((0, 0), (0, 0), (0, S64 - NUM_SAMPLES)),
                    mode="edge")
    idx_flat = idx_p.reshape(N_GATHER)
    w_t = jnp.transpose(weights, (0, 2, 1))
    w_p = jnp.pad(w_t, ((0, 0), (0, 0), (0, S64 - NUM_SAMPLES)),
                  constant_values=1.0)
    w16 = w_p.reshape(N_GROWS, 16)
    P = _project(table)
    P2 = P.reshape(N_PROWS * 128).reshape(NUM_VOXELS, PROJ_DIM)
    G = _sc_gather(P2, idx_flat)
    G128 = G.reshape(N_GATHER * PROJ_DIM).reshape(N_GROWS, 128)
    return _fused(G128, w16)


# speedup vs baseline: 1.0107x; 1.0107x over previous
import math

import jax
import jax.numpy as jnp
import numpy as np
from jax import lax
from jax.experimental import pallas as pl
from jax.experimental.pallas import tpu as pltpu
from jax.experimental.pallas import tpu_sc as plsc

GRID_RES = 128
FEATURE_DIM = 28
NUM_VOXELS = GRID_RES ** 3
NUM_RAYS = 4096
NUM_SAMPLES = 50
RAY_LENGTH = 100.0
DELTA = RAY_LENGTH / NUM_SAMPLES
THETA = 0.5
PHI = 0.3

PROJ_DIM = 8
NUM_NBR = 8
S64 = 64

VPR = 16
N_PROWS = NUM_VOXELS // VPR

N_GATHER = NUM_RAYS * NUM_NBR * S64
N_GROWS = N_GATHER // 16

NUM_WORKERS = 32
PER_W = N_GATHER // NUM_WORKERS
CHUNK = 4096
N_CHUNKS = PER_W // CHUNK


def _sh_vec():
    y00 = 0.5 * math.sqrt(1.0 / math.pi)
    h3 = 0.5 * math.sqrt(3.0 / math.pi)
    q5 = 0.25 * math.sqrt(5.0 / math.pi)
    h15 = 0.5 * math.sqrt(15.0 / math.pi)
    q15 = 0.25 * math.sqrt(15.0 / math.pi)
    st, ct = math.sin(THETA), math.cos(THETA)
    sp, cp = math.sin(PHI), math.cos(PHI)
    return np.array([
        y00,
        h3 * st * sp,
        h3 * ct,
        h3 * st * cp,
        h15 * st * cp * st * sp,
        h15 * st * sp * ct,
        q5 * (3.0 * ct * ct - 1.0),
        h15 * st * cp * ct,
        q15 * ((st * cp) ** 2 - (st * sp) ** 2),
    ], dtype=np.float32)


def _proj_matrix():
    Y = _sh_vec()
    M = np.zeros((FEATURE_DIM, PROJ_DIM), dtype=np.float32)
    M[0, 0] = 1.0
    for c in range(3):
        M[1 + c * 9:1 + (c + 1) * 9, 1 + c] = Y
    return M


def _proj_body(t_ref, m_ref, b_ref, o_ref):
    o_ref[...] = jnp.dot(t_ref[...], m_ref[...],
                         preferred_element_type=jnp.float32) + b_ref[...]


def _project(table):
    M = _proj_matrix()
    D = np.zeros((VPR * FEATURE_DIM, 128), dtype=np.float32)
    for j in range(VPR):
        D[j * FEATURE_DIM:(j + 1) * FEATURE_DIM,
          j * PROJ_DIM:j * PROJ_DIM + PROJ_DIM] = M
    bias = np.zeros((1, 128), dtype=np.float32)
    for j in range(VPR):
        bias[0, j * PROJ_DIM + 7] = 1.0
    t448 = table.reshape(N_PROWS, VPR * FEATURE_DIM)
    BLK = 4096
    grid = N_PROWS // BLK
    return pl.pallas_call(
        _proj_body,
        grid=(grid,),
        in_specs=[
            pl.BlockSpec((BLK, VPR * FEATURE_DIM), lambda i: (i, 0)),
            pl.BlockSpec((VPR * FEATURE_DIM, 128), lambda i: (0, 0)),
            pl.BlockSpec((1, 128), lambda i: (0, 0)),
        ],
        out_specs=pl.BlockSpec((BLK, 128), lambda i: (i, 0)),
        out_shape=jax.ShapeDtypeStruct((N_PROWS, 128), jnp.float32),
    )(t448, jnp.asarray(D), jnp.asarray(bias))


def _gather_body(p_hbm, idx_hbm, out_hbm,
                 idx_v0, idx_v1, rows_v0, rows_v1, sem0, sem1):
    wid = lax.axis_index("s") * 2 + lax.axis_index("c")
    idx_v = (idx_v0, idx_v1)
    rows_v = (rows_v0, rows_v1)
    sems = (sem0, sem1)
    base0 = wid * PER_W
    pltpu.sync_copy(idx_hbm.at[pl.ds(base0, CHUNK)], idx_v[0])
    handles = [pltpu.async_copy(p_hbm.at[idx_v[0]], rows_v[0], sems[0])]
    for c in range(1, N_CHUNKS + 1):
        if c < N_CHUNKS:
            b = c % 2
            base = wid * PER_W + c * CHUNK
            pltpu.sync_copy(idx_hbm.at[pl.ds(base, CHUNK)], idx_v[b])
            handles.append(
                pltpu.async_copy(p_hbm.at[idx_v[b]], rows_v[b], sems[b]))
        pb = (c - 1) % 2
        pbase = wid * PER_W + (c - 1) * CHUNK
        handles[c - 1].wait()
        pltpu.sync_copy(rows_v[pb], out_hbm.at[pl.ds(pbase, CHUNK)])


def _sc_gather(P2, idx_flat):
    mesh = plsc.VectorSubcoreMesh(core_axis_name="c", subcore_axis_name="s")
    k = pl.kernel(
        _gather_body,
        mesh=mesh,
        out_type=jax.ShapeDtypeStruct((N_GATHER, PROJ_DIM), jnp.float32),
        scratch_types=[
            pltpu.VMEM((CHUNK,), jnp.int32),
            pltpu.VMEM((CHUNK,), jnp.int32),
            pltpu.VMEM((CHUNK, PROJ_DIM), jnp.float32),
            pltpu.VMEM((CHUNK, PROJ_DIM), jnp.float32),
            pltpu.SemaphoreType.DMA,
            pltpu.SemaphoreType.DMA,
        ],
        compiler_params=pltpu.CompilerParams(use_tc_tiling_on_sc=False),
    )
    return k(P2, idx_flat)


R_BLK = 128
BR = R_BLK * 32


def _fused_consts():
    E = np.zeros((16, 128), dtype=np.float32)
    for t in range(16):
        E[t, t * 8:t * 8 + 8] = 1.0
    SPR7 = np.zeros((128, 128), dtype=np.float32)
    for t in range(16):
        for f in range(8):
            SPR7[t * 8 + 7, t * 8 + f] = 1.0
    LTIN = np.zeros((128, 128), dtype=np.float32)
    for t in range(16):
        for t2 in range(t, 16):
            LTIN[t * 8, t2 * 8] = 1.0
    TOTM = np.zeros((128, 128), dtype=np.float32)
    for t in range(16):
        for t2 in range(16):
            TOTM[t * 8, t2 * 8] = 1.0
    SPRD = np.zeros((128, 128), dtype=np.float32)
    for t in range(16):
        for c in (1, 2, 3):
            SPRD[t * 8, t * 8 + c] = 1.0
    UC = np.zeros((128, 3), dtype=np.float32)
    for t in range(16):
        for c in range(3):
            UC[t * 8 + 1 + c, c] = 1.0
    SEL = np.zeros((R_BLK, BR), dtype=np.float32)
    for r in range(R_BLK):
        for q in range(4):
            SEL[r, r * 32 + q] = 1.0
    return E, SPR7, LTIN, TOTM, SPRD, UC, SEL


def _fused_body(g_ref, w_ref, e_ref, spr7_ref, ltin_ref, totm_ref,
                sprd_ref, uc_ref, sel_ref, o_ref):
    g = g_ref[...]
    w = w_ref[...]
    wexp = jnp.dot(w, e_ref[...], preferred_element_type=jnp.float32)
    acc = g * wexp
    acc = acc + pltpu.roll(acc, BR - 4, axis=0)
    acc = acc + pltpu.roll(acc, BR - 8, axis=0)
    acc = acc + pltpu.roll(acc, BR - 16, axis=0)
    nrm = jnp.dot(acc, spr7_ref[...], preferred_element_type=jnp.float32)
    z = acc / nrm
    rowi = lax.broadcasted_iota(jnp.int32, (BR, 128), 0)
    lanei = lax.broadcasted_iota(jnp.int32, (BR, 128), 1)
    dmask = (lanei % 8 == 0).astype(jnp.float32)
    taud = jnp.maximum(z, 0.0) * dmask * DELTA
    pre = jnp.dot(taud, ltin_ref[...], preferred_element_type=jnp.float32)
    excl = pre - taud
    tot = jnp.dot(taud, totm_ref[...], preferred_element_type=jnp.float32)
    base = jnp.zeros_like(tot)
    for k in (1, 2, 3):
        mk = (rowi % 32 >= k).astype(jnp.float32)
        base = base + pltpu.roll(tot, k, axis=0) * mk
    T = jnp.exp(-(excl + base))
    alpha = 1.0 - jnp.exp(-taud)
    wta = T * alpha * dmask
    wta8 = jnp.dot(wta, sprd_ref[...], preferred_element_type=jnp.float32)
    smask = ((rowi % 32) * 16 + lanei // 8 < NUM_SAMPLES).astype(jnp.float32)
    contrib = z * wta8 * smask
    ruc = jnp.dot(contrib, uc_ref[...], preferred_element_type=jnp.float32)
    o_ref[...] = jnp.dot(sel_ref[...], ruc,
                         preferred_element_type=jnp.float32)


def _fused(G128, w16):
    E, SPR7, LTIN, TOTM, SPRD, UC, SEL = _fused_consts()
    grid = NUM_RAYS // R_BLK
    return pl.pallas_call(
        _fused_body,
        grid=(grid,),
        in_specs=[
            pl.BlockSpec((BR, 128), lambda i: (i, 0)),
            pl.BlockSpec((BR, 16), lambda i: (i, 0)),
            pl.BlockSpec((16, 128), lambda i: (0, 0)),
            pl.BlockSpec((128, 128), lambda i: (0, 0)),
            pl.BlockSpec((128, 128), lambda i: (0, 0)),
            pl.BlockSpec((128, 128), lambda i: (0, 0)),
            pl.BlockSpec((128, 128), lambda i: (0, 0)),
            pl.BlockSpec((128, 3), lambda i: (0, 0)),
            pl.BlockSpec((R_BLK, BR), lambda i: (0, 0)),
        ],
        out_specs=pl.BlockSpec((R_BLK, 3), lambda i: (i, 0)),
        out_shape=jax.ShapeDtypeStruct((NUM_RAYS, 3), jnp.float32),
    )(G128, w16, jnp.asarray(E), jnp.asarray(SPR7), jnp.asarray(LTIN),
      jnp.asarray(TOTM), jnp.asarray(SPRD), jnp.asarray(UC), jnp.asarray(SEL))


def kernel(weights, table, indices):
    idx_t = jnp.transpose(indices.astype(jnp.int32), (0, 2, 1))
    idx_p = jnp.pad(idx_t, ((0, 0), (0, 0), (0, S64 - NUM_SAMPLES)),
                    mode="edge")
    idx_flat = idx_p.reshape(N_GATHER)
    w_t = jnp.transpose(weights, (0, 2, 1))
    w_p = jnp.pad(w_t, ((0, 0), (0, 0), (0, S64 - NUM_SAMPLES)),
                  constant_values=1.0)
    w16 = w_p.reshape(N_GROWS, 16)
    P = _project(table)
    P2 = P.reshape(N_PROWS * 128).reshape(NUM_VOXELS, PROJ_DIM)
    G = _sc_gather(P2, idx_flat)
    G128 = G.reshape(N_GATHER * PROJ_DIM).reshape(N_GROWS, 128)
    return _fused(G128, w16)
